# baseline (speedup 1.0000x reference)
import functools

import jax
import jax.numpy as jnp
from jax.experimental import pallas as pl
from jax.experimental.pallas import tpu as pltpu

_C = 16
_R = 64


def _gemm_kernel(x_hbm, w_ref, b_ref, o_hbm, xbuf, obuf, in_sem, out_sem,
                 *, nsteps: int, tb: int, depth: int):
    tc = pl.program_id(0)
    base = tc * nsteps

    def start_in(slot, step):
        pltpu.make_async_copy(
            x_hbm.at[pl.ds((base + step) * tb, tb), :],
            xbuf.at[slot], in_sem.at[slot]).start()

    def wait_in(slot):
        pltpu.make_async_copy(xbuf.at[slot], xbuf.at[slot],
                              in_sem.at[slot]).wait()

    def start_out(slot, c, step):
        pltpu.make_async_copy(
            obuf.at[slot, c],
            o_hbm.at[pl.ds((base + step) * tb, tb), c],
            out_sem.at[slot, c]).start()

    def wait_out(slot, c):
        pltpu.make_async_copy(obuf.at[slot, c], obuf.at[slot, c],
                              out_sem.at[slot, c]).wait()

    start_in(0, 0)

    def body(step, _):
        cur = jax.lax.rem(step, 2)
        o_slot = jax.lax.rem(step, depth)

        @pl.when(step + 1 < nsteps)
        def _():
            start_in(jax.lax.rem(step + 1, 2), step + 1)

        wait_in(cur)
        xb = xbuf[cur].astype(jnp.bfloat16)

        for c in range(_C):
            @pl.when(step >= depth)
            def _():
                wait_out(o_slot, c)

            ob = obuf.at[o_slot, c]
            ob[...] = jnp.dot(xb, w_ref[c],
                              preferred_element_type=jnp.float32) + b_ref[c]
            start_out(o_slot, c, step)
        return ()

    jax.lax.fori_loop(0, nsteps, body, ())

    tail = min(depth, nsteps)
    for d in range(tail):
        for c in range(_C):
            wait_out((nsteps - tail + d) % depth, c)


@functools.partial(jax.jit, static_argnames=("tb", "depth"))
def _forward(x, w_main, w_bias, *, tb, depth):
    B, V = x.shape
    N = w_main.shape[1]
    assert N == _C * _R and B % (2 * tb) == 0
    nsteps = B // (2 * tb)

    w3 = w_main.astype(jnp.bfloat16).reshape(V, _C, _R).transpose(1, 0, 2)
    b3 = w_bias.reshape(1, _C, _R).transpose(1, 0, 2)

    return pl.pallas_call(
        functools.partial(_gemm_kernel, nsteps=nsteps, tb=tb, depth=depth),
        out_shape=jax.ShapeDtypeStruct((B, _C, _R), jnp.float32),
        grid=(2,),
        in_specs=[
            pl.BlockSpec(memory_space=pl.ANY),
            pl.BlockSpec((_C, V, _R), lambda i: (0, 0, 0)),
            pl.BlockSpec((_C, 1, _R), lambda i: (0, 0, 0)),
        ],
        out_specs=pl.BlockSpec(memory_space=pl.ANY),
        scratch_shapes=[
            pltpu.VMEM((2, tb, V), jnp.float32),
            pltpu.VMEM((depth, _C, tb, _R), jnp.float32),
            pltpu.SemaphoreType.DMA((2,)),
            pltpu.SemaphoreType.DMA((depth, _C)),
        ],
        compiler_params=pltpu.CompilerParams(
            dimension_semantics=("parallel",),
            vmem_limit_bytes=64 * 1024 * 1024,
        ),
        cost_estimate=pl.CostEstimate(
            flops=2 * B * N * V,
            transcendentals=0,
            bytes_accessed=4 * (B * V + B * N) + 2 * V * N,
        ),
    )(x, w3, b3)


def kernel(x, w_main, w_bias):
    return _forward(x, w_main, w_bias, tb=1024, depth=2)


# speedup vs baseline: 1.1741x; 1.1741x over previous
import functools

import jax
import jax.numpy as jnp
from jax.experimental import pallas as pl
from jax.experimental.pallas import tpu as pltpu

_C = 16
_R = 64


def _gemm_kernel(x_hbm, w_ref, b_ref, o_hbm, xbuf, obuf, in_sem, out_sem,
                 *, nsteps: int, tb: int, depth: int):
    tc = pl.program_id(0)
    base = tc * nsteps

    def start_in(slot, step):
        pltpu.make_async_copy(
            x_hbm.at[pl.ds((base + step) * tb, tb), :],
            xbuf.at[slot], in_sem.at[slot]).start()

    def wait_in(slot):
        pltpu.make_async_copy(xbuf.at[slot], xbuf.at[slot],
                              in_sem.at[slot]).wait()

    def start_out(slot, step):
        pltpu.make_async_copy(
            obuf.at[slot],
            o_hbm.at[pl.ds((base + step) * tb, tb), :, :],
            out_sem.at[slot]).start()

    def wait_out(slot):
        pltpu.make_async_copy(obuf.at[slot], obuf.at[slot],
                              out_sem.at[slot]).wait()

    start_in(0, 0)

    def body(step, _):
        cur = jax.lax.rem(step, 2)
        o_slot = jax.lax.rem(step, depth)

        @pl.when(step + 1 < nsteps)
        def _():
            start_in(jax.lax.rem(step + 1, 2), step + 1)

        wait_in(cur)

        @pl.when(step >= depth)
        def _():
            wait_out(o_slot)

        xb = xbuf[cur].astype(jnp.bfloat16)
        acc = jnp.dot(xb, w_ref[...],
                      preferred_element_type=jnp.float32) + b_ref[...]
        ob = obuf.at[o_slot]
        ob[...] = acc.reshape(tb, _C, _R)
        start_out(o_slot, step)
        return ()

    jax.lax.fori_loop(0, nsteps, body, ())

    tail = min(depth, nsteps)
    for d in range(tail):
        wait_out((nsteps - tail + d) % depth)


@functools.partial(jax.jit, static_argnames=("tb", "depth"))
def _forward(x, w_main, w_bias, *, tb, depth):
    B, V = x.shape
    N = w_main.shape[1]
    assert N == _C * _R and B % (2 * tb) == 0
    nsteps = B // (2 * tb)
    wb = w_main.astype(jnp.bfloat16)

    return pl.pallas_call(
        functools.partial(_gemm_kernel, nsteps=nsteps, tb=tb, depth=depth),
        out_shape=jax.ShapeDtypeStruct((B, _C, _R), jnp.float32),
        grid=(2,),
        in_specs=[
            pl.BlockSpec(memory_space=pl.ANY),
            pl.BlockSpec((V, N), lambda i: (0, 0)),
            pl.BlockSpec((1, N), lambda i: (0, 0)),
        ],
        out_specs=pl.BlockSpec(memory_space=pl.ANY),
        scratch_shapes=[
            pltpu.VMEM((2, tb, V), jnp.float32),
            pltpu.VMEM((depth, tb, _C, _R), jnp.float32),
            pltpu.SemaphoreType.DMA((2,)),
            pltpu.SemaphoreType.DMA((depth,)),
        ],
        compiler_params=pltpu.CompilerParams(
            dimension_semantics=("parallel",),
            vmem_limit_bytes=64 * 1024 * 1024,
        ),
        cost_estimate=pl.CostEstimate(
            flops=2 * B * N * V,
            transcendentals=0,
            bytes_accessed=4 * (B * V + 2 * B * N) + 2 * V * N,
        ),
    )(x, wb, w_bias)


def kernel(x, w_main, w_bias):
    return _forward(x, w_main, w_bias, tb=1024, depth=2)


# speedup vs baseline: 5.5273x; 4.7076x over previous
import functools

import jax
import jax.numpy as jnp
from jax.experimental import pallas as pl
from jax.experimental.pallas import tpu as pltpu

_C = 16
_R = 64


def _gemm_t_kernel(x_ref, w_ref, b_ref, o_ref, *, tb: int):
    xb = x_ref[...].astype(jnp.bfloat16)
    acc = jax.lax.dot_general(w_ref[...], xb, (((0,), (1,)), ((), ())),
                              preferred_element_type=jnp.float32)
    o_ref[...] = (acc + b_ref[...]).reshape(_C, _R, tb)


@functools.partial(jax.jit, static_argnames=("tb",))
def _forward(x, w_main, w_bias, *, tb):
    B, V = x.shape
    N = w_main.shape[1]
    assert N == _C * _R and B % tb == 0
    wb = w_main.astype(jnp.bfloat16)
    bt = w_bias.reshape(N, 1)

    out_t = pl.pallas_call(
        functools.partial(_gemm_t_kernel, tb=tb),
        out_shape=jax.ShapeDtypeStruct((_C, _R, B), jnp.float32),
        grid=(B // tb,),
        in_specs=[
            pl.BlockSpec((tb, V), lambda i: (i, 0)),
            pl.BlockSpec((V, N), lambda i: (0, 0)),
            pl.BlockSpec((N, 1), lambda i: (0, 0)),
        ],
        out_specs=pl.BlockSpec((_C, _R, tb), lambda i: (0, 0, i)),
        compiler_params=pltpu.CompilerParams(
            dimension_semantics=("parallel",),
            vmem_limit_bytes=64 * 1024 * 1024,
        ),
        cost_estimate=pl.CostEstimate(
            flops=2 * B * N * V,
            transcendentals=0,
            bytes_accessed=4 * (B * V + B * N) + 2 * V * N,
        ),
    )(x, wb, bt)
    return out_t.transpose(2, 0, 1)


def kernel(x, w_main, w_bias):
    return _forward(x, w_main, w_bias, tb=1024)


# speedup vs baseline: 6.2570x; 1.1320x over previous
import functools

import jax
import jax.numpy as jnp
from jax.experimental import pallas as pl
from jax.experimental.pallas import tpu as pltpu

_C = 16
_R = 64


def _gemm_t_kernel(x_ref, w_ref, b_ref, o_ref, *, tb: int):
    xb = x_ref[...].astype(jnp.bfloat16)
    acc = jax.lax.dot_general(w_ref[...], xb, (((0,), (1,)), ((), ())),
                              preferred_element_type=jnp.float32)
    o_ref[...] = (acc + b_ref[...]).reshape(_C, _R, tb)


@functools.partial(jax.jit, static_argnames=("tb",))
def _forward(x, w_main, w_bias, *, tb):
    B, V = x.shape
    N = w_main.shape[1]
    assert N == _C * _R and B % tb == 0
    wb = w_main.astype(jnp.bfloat16)
    bt = w_bias.reshape(N, 1)

    out_t = pl.pallas_call(
        functools.partial(_gemm_t_kernel, tb=tb),
        out_shape=jax.ShapeDtypeStruct((_C, _R, B), jnp.float32),
        grid=(B // tb,),
        in_specs=[
            pl.BlockSpec((tb, V), lambda i: (i, 0)),
            pl.BlockSpec((V, N), lambda i: (0, 0)),
            pl.BlockSpec((N, 1), lambda i: (0, 0)),
        ],
        out_specs=pl.BlockSpec((_C, _R, tb), lambda i: (0, 0, i)),
        compiler_params=pltpu.CompilerParams(
            dimension_semantics=("parallel",),
            vmem_limit_bytes=64 * 1024 * 1024,
        ),
        cost_estimate=pl.CostEstimate(
            flops=2 * B * N * V,
            transcendentals=0,
            bytes_accessed=4 * (B * V + B * N) + 2 * V * N,
        ),
    )(x, wb, bt)
    return out_t.transpose(2, 0, 1)


def kernel(x, w_main, w_bias):
    return _forward(x, w_main, w_bias, tb=2048)


# speedup vs baseline: 6.3219x; 1.0104x over previous
import functools

import jax
import jax.numpy as jnp
from jax.experimental import pallas as pl
from jax.experimental.pallas import tpu as pltpu

_C = 16
_R = 64


def _gemm_t_kernel(x_ref, w_ref, b_ref, o_ref, *, tb: int):
    xb = x_ref[...].astype(jnp.bfloat16)
    acc = jax.lax.dot_general(w_ref[...], xb, (((0,), (1,)), ((), ())),
                              preferred_element_type=jnp.float32)
    o_ref[...] = (acc + b_ref[...]).reshape(_C, _R, tb)


@functools.partial(jax.jit, static_argnames=("tb",))
def _forward(x, w_main, w_bias, *, tb):
    B, V = x.shape
    N = w_main.shape[1]
    assert N == _C * _R and B % tb == 0
    wb = w_main.astype(jnp.bfloat16)
    bt = w_bias.reshape(N, 1)

    out_t = pl.pallas_call(
        functools.partial(_gemm_t_kernel, tb=tb),
        out_shape=jax.ShapeDtypeStruct((_C, _R, B), jnp.float32),
        grid=(B // tb,),
        in_specs=[
            pl.BlockSpec((tb, V), lambda i: (i, 0)),
            pl.BlockSpec((V, N), lambda i: (0, 0)),
            pl.BlockSpec((N, 1), lambda i: (0, 0)),
        ],
        out_specs=pl.BlockSpec((_C, _R, tb), lambda i: (0, 0, i)),
        compiler_params=pltpu.CompilerParams(
            dimension_semantics=("parallel",),
            vmem_limit_bytes=64 * 1024 * 1024,
        ),
        cost_estimate=pl.CostEstimate(
            flops=2 * B * N * V,
            transcendentals=0,
            bytes_accessed=4 * (B * V + B * N) + 2 * V * N,
        ),
    )(x, wb, bt)
    return out_t.transpose(2, 0, 1)


def kernel(x, w_main, w_bias):
    return _forward(x, w_main, w_bias, tb=4096)


# speedup vs baseline: 6.8717x; 1.0870x over previous
import functools

import jax
import jax.numpy as jnp
from jax.experimental import pallas as pl
from jax.experimental.pallas import tpu as pltpu

_C = 16
_R = 64


def _gemm_t_kernel(x_ref, w_ref, b_ref, o_ref, *, tb: int):
    xb = x_ref[...].astype(jnp.bfloat16)
    wb = w_ref[...].astype(jnp.bfloat16)
    acc = jax.lax.dot_general(wb, xb, (((0,), (1,)), ((), ())),
                              preferred_element_type=jnp.float32)
    bias = b_ref[...].reshape(_C * _R, 1)
    o_ref[...] = (acc + bias).reshape(_C, _R, tb)


@functools.partial(jax.jit, static_argnames=("tb",))
def _forward(x, w_main, w_bias, *, tb):
    B, V = x.shape
    N = w_main.shape[1]
    assert N == _C * _R and B % tb == 0
    out_t = pl.pallas_call(
        functools.partial(_gemm_t_kernel, tb=tb),
        out_shape=jax.ShapeDtypeStruct((_C, _R, B), jnp.float32),
        grid=(B // tb,),
        in_specs=[
            pl.BlockSpec((tb, V), lambda i: (i, 0)),
            pl.BlockSpec((V, N), lambda i: (0, 0)),
            pl.BlockSpec((1, N), lambda i: (0, 0)),
        ],
        out_specs=pl.BlockSpec((_C, _R, tb), lambda i: (0, 0, i)),
        compiler_params=pltpu.CompilerParams(
            dimension_semantics=("parallel",),
            vmem_limit_bytes=64 * 1024 * 1024,
        ),
        cost_estimate=pl.CostEstimate(
            flops=2 * B * N * V,
            transcendentals=0,
            bytes_accessed=4 * (B * V + B * N) + 2 * V * N,
        ),
    )(x, w_main, w_bias)
    return out_t.transpose(2, 0, 1)


def kernel(x, w_main, w_bias):
    return _forward(x, w_main, w_bias, tb=4096)
